# baseline (speedup 1.0000x reference)
import jax
import jax.numpy as jnp
from jax.experimental import pallas as pl

NUM_USERS = 25000
NUM_ITEMS = 25000
DIM = 64
NUM_LAYERS = 3
N = NUM_USERS + NUM_ITEMS


def _mean4_body(a_ref, b_ref, c_ref, d_ref, o_ref):
    o_ref[...] = (a_ref[...] + b_ref[...] + c_ref[...] + d_ref[...]) * 0.25


def kernel(adj_indices, adj_values, user_emb, item_emb):
    all_embs = jnp.concatenate([user_emb, item_emb], axis=0)
    row, col = adj_indices[0], adj_indices[1]
    embs = [all_embs]
    x = all_embs
    for _ in range(NUM_LAYERS):
        g = jnp.take(x, col, axis=0) * adj_values[:, None]
        x = jax.ops.segment_sum(g, row, num_segments=N)
        embs.append(x)
    blk = 2000
    grid = (N // blk,)
    spec = pl.BlockSpec((blk, DIM), lambda i: (i, 0))
    out = pl.pallas_call(
        _mean4_body,
        grid=grid,
        in_specs=[spec] * 4,
        out_specs=spec,
        out_shape=jax.ShapeDtypeStruct((N, DIM), jnp.float32),
    )(*embs)
    return out[:NUM_USERS], out[NUM_USERS:]


# speedup vs baseline: 3.6807x; 3.6807x over previous
import functools

import jax
import jax.numpy as jnp
from jax import lax
from jax.experimental import pallas as pl
from jax.experimental.pallas import tpu as pltpu
from jax.experimental.pallas import tpu_sc as plsc

NUM_USERS = 25000
NUM_ITEMS = 25000
DIM = 64
NUM_LAYERS = 3
E = 800000
N = NUM_USERS + NUM_ITEMS

HALF = 25000
PAD = 88
HP = HALF + PAD
NP = 2 * HP
CHUNK = 128
NCH_TOT = 6272
NCH_SUB = NCH_TOT // 16
PADE = NCH_TOT * CHUNK - E
STRIPE = HP // 16
ZR = 98

_MESH = plsc.VectorSubcoreMesh(core_axis_name="c", subcore_axis_name="s")
_GATHER_DNUMS = lax.GatherDimensionNumbers(
    offset_dims=(), collapsed_slice_dims=(0,), start_index_map=(0,))


def _layer(emb, rows2d, cols2d, vals2d):
    @functools.partial(
        pl.kernel,
        out_type=jax.ShapeDtypeStruct((NP, DIM), jnp.float32),
        mesh=_MESH,
        compiler_params=pltpu.CompilerParams(use_tc_tiling_on_sc=False),
        scratch_types=[
            pltpu.VMEM((1, CHUNK), jnp.int32),
            pltpu.VMEM((1, CHUNK), jnp.int32),
            pltpu.VMEM((1, CHUNK), jnp.int32),
            pltpu.VMEM((1, CHUNK), jnp.int32),
            pltpu.VMEM((1, CHUNK), jnp.float32),
            pltpu.VMEM((1, CHUNK), jnp.float32),
            pltpu.VMEM((CHUNK, DIM), jnp.float32),
            pltpu.VMEM((CHUNK, DIM), jnp.float32),
            pltpu.VMEM((ZR, DIM), jnp.float32),
            pltpu.VMEM_SHARED((HP, DIM), jnp.float32),
            pltpu.SemaphoreType.DMA,
            pltpu.SemaphoreType.DMA,
        ],
    )
    def body(emb_hbm, rows_hbm, cols_hbm, vals_hbm, out_hbm,
             rbuf0, rbuf1, cbuf0, cbuf1, vbuf0, vbuf1,
             gbuf0, gbuf1, zbuf, acc, sem0, sem1):
        cid = lax.axis_index("c")
        sid = lax.axis_index("s")
        base = cid * HALF
        cbase = sid * NCH_SUB
        lane = lax.broadcasted_iota(jnp.int32, (16,), 0)
        junk16 = HALF + ((sid * 5 + lane) & 63)

        zv = jnp.zeros((16,), jnp.float32)

        @pl.loop(0, ZR)
        def _(r):
            for q in range(4):
                zbuf[r, pl.ds(q * 16, 16)] = zv

        @pl.loop(0, 16)
        def _(k):
            pltpu.sync_copy(zbuf, acc.at[pl.ds(sid * STRIPE + k * ZR, ZR), :])

        plsc.subcore_barrier()

        def prep(rbuf, cbuf, vbuf, gbuf, sem, ci):
            c = cbase + ci
            pltpu.sync_copy(rows_hbm.at[pl.ds(c, 1), :], rbuf)
            pltpu.sync_copy(cols_hbm.at[pl.ds(c, 1), :], cbuf)
            pltpu.sync_copy(vals_hbm.at[pl.ds(c, 1), :], vbuf)

            @pl.loop(0, 8)
            def _(g):
                sl = pl.ds(g * 16, 16)
                col = cbuf[0, sl]
                cbuf[0, sl] = col + jnp.where(col >= NUM_USERS, PAD, 0)
                r = rbuf[0, sl]
                inhalf = (r >= base) & (r < base + HALF)
                rbuf[0, sl] = jnp.where(inhalf, r - base, junk16)

            pltpu.async_copy(emb_hbm.at[cbuf.at[0]], gbuf, sem)

        def work(rbuf, cbuf, vbuf, gbuf, sem):
            pltpu.make_async_copy(emb_hbm.at[cbuf.at[0]], gbuf, sem).wait()

            @pl.loop(0, 8)
            def _(g):
                v16 = vbuf[0, pl.ds(g * 16, 16)]
                for j in range(16):
                    b = lax.gather(
                        v16, jnp.full((16, 1), j, jnp.int32),
                        _GATHER_DNUMS, slice_sizes=(1,),
                        mode=lax.GatherScatterMode.PROMISE_IN_BOUNDS)
                    e = g * 16 + j
                    for q in range(4):
                        sl = pl.ds(q * 16, 16)
                        gbuf[e, sl] = gbuf[e, sl] * b

            pltpu.sync_copy(gbuf, acc.at[rbuf.at[0]], add=True)

        prep(rbuf0, cbuf0, vbuf0, gbuf0, sem0, 0)
        prep(rbuf1, cbuf1, vbuf1, gbuf1, sem1, 1)

        @pl.loop(0, NCH_SUB // 2 - 1)
        def _(p):
            work(rbuf0, cbuf0, vbuf0, gbuf0, sem0)
            prep(rbuf0, cbuf0, vbuf0, gbuf0, sem0, 2 * p + 2)
            work(rbuf1, cbuf1, vbuf1, gbuf1, sem1)
            prep(rbuf1, cbuf1, vbuf1, gbuf1, sem1, 2 * p + 3)

        work(rbuf0, cbuf0, vbuf0, gbuf0, sem0)
        work(rbuf1, cbuf1, vbuf1, gbuf1, sem1)

        plsc.subcore_barrier()
        pltpu.sync_copy(
            acc.at[pl.ds(sid * STRIPE, STRIPE), :],
            out_hbm.at[pl.ds(cid * HP + sid * STRIPE, STRIPE), :])

    return body(emb, rows2d, cols2d, vals2d)


def _mean4_body(a_ref, b_ref, c_ref, d_ref, o_ref):
    o_ref[...] = (a_ref[...] + b_ref[...] + c_ref[...] + d_ref[...]) * 0.25


def _mean4(e0, e1, e2, e3):
    blk = STRIPE
    spec = pl.BlockSpec((blk, DIM), lambda i: (i, 0))
    return pl.pallas_call(
        _mean4_body,
        grid=(NP // blk,),
        in_specs=[spec] * 4,
        out_specs=spec,
        out_shape=jax.ShapeDtypeStruct((NP, DIM), jnp.float32),
    )(e0, e1, e2, e3)


def kernel(adj_indices, adj_values, user_emb, item_emb):
    rows = adj_indices[0].astype(jnp.int32)
    cols = adj_indices[1].astype(jnp.int32)
    rows2d = jnp.pad(rows, (0, PADE)).reshape(NCH_TOT, CHUNK)
    cols2d = jnp.pad(cols, (0, PADE)).reshape(NCH_TOT, CHUNK)
    vals2d = jnp.pad(adj_values, (0, PADE)).reshape(NCH_TOT, CHUNK)
    zpad = jnp.zeros((PAD, DIM), jnp.float32)
    e0 = jnp.concatenate([user_emb, zpad, item_emb, zpad], axis=0)
    e1 = _layer(e0, rows2d, cols2d, vals2d)
    e2 = _layer(e1, rows2d, cols2d, vals2d)
    e3 = _layer(e2, rows2d, cols2d, vals2d)
    m = _mean4(e0, e1, e2, e3)
    return m[:NUM_USERS], m[HP:HP + NUM_ITEMS]
